# speedup vs baseline: 1.6208x; 1.6208x over previous
import functools

import jax
import jax.numpy as jnp
from jax import lax
from jax.experimental import pallas as pl
from jax.experimental.pallas import tpu as pltpu
from jax.experimental.pallas import tpu_sc as plsc

_CHUNK = 64


def _embedding_lookup(tokens_flat, table):
    B, = tokens_flat.shape
    V, D = table.shape
    info = plsc.get_sparse_core_info()
    NC, NS = info.num_cores, info.num_subcores
    NW = NC * NS
    b_per_w = B // NW
    n_chunks = b_per_w // _CHUNK
    assert B == NW * b_per_w and b_per_w == n_chunks * _CHUNK

    idx2d = tokens_flat.reshape(B // _CHUNK, _CHUNK)
    mesh = plsc.VectorSubcoreMesh(core_axis_name="c", subcore_axis_name="s")

    @functools.partial(
        pl.kernel,
        mesh=mesh,
        out_type=jax.ShapeDtypeStruct((B, D), jnp.float32),
        scratch_types=[
            pltpu.VMEM((n_chunks, _CHUNK), jnp.int32),
            pltpu.VMEM((_CHUNK, D), jnp.float32),
            pltpu.SemaphoreType.DMA,
        ],
    )
    def gather_kernel(idx_hbm, table_hbm, out_hbm, idx_v, rows_v, sem):
        wid = lax.axis_index("s") * NC + lax.axis_index("c")
        base_chunk = wid * n_chunks
        pltpu.sync_copy(idx_hbm.at[pl.ds(base_chunk, n_chunks)], idx_v)

        def body(i, carry):
            pltpu.async_copy(table_hbm.at[idx_v.at[i]], rows_v, sem).wait()
            row0 = (base_chunk + i) * _CHUNK
            pltpu.sync_copy(rows_v, out_hbm.at[pl.ds(row0, _CHUNK)])
            return carry

        lax.fori_loop(0, n_chunks, body, 0)

    return gather_kernel(idx2d, table)


def kernel(tokens, start_pos, tok_embeddings_weight):
    B, S = tokens.shape
    V, D = tok_embeddings_weight.shape
    out = _embedding_lookup(tokens.reshape(B * S), tok_embeddings_weight)
    return out.reshape(B, S, D)


# speedup vs baseline: 1.7097x; 1.0548x over previous
import functools

import jax
import jax.numpy as jnp
from jax import lax
from jax.experimental import pallas as pl
from jax.experimental.pallas import tpu as pltpu
from jax.experimental.pallas import tpu_sc as plsc

_CHUNK = 32


def _embedding_lookup(tokens_flat, table):
    B, = tokens_flat.shape
    V, D = table.shape
    info = plsc.get_sparse_core_info()
    NC, NS = info.num_cores, info.num_subcores
    NW = NC * NS
    b_per_w = B // NW
    n_chunks = b_per_w // _CHUNK
    assert B == NW * b_per_w and b_per_w == n_chunks * _CHUNK

    idx2d = tokens_flat.reshape(B // _CHUNK, _CHUNK)
    mesh = plsc.VectorSubcoreMesh(core_axis_name="c", subcore_axis_name="s")

    @functools.partial(
        pl.kernel,
        mesh=mesh,
        out_type=jax.ShapeDtypeStruct((B, D), jnp.float32),
        scratch_types=[
            pltpu.VMEM((n_chunks, _CHUNK), jnp.int32),
            pltpu.VMEM((_CHUNK, D), jnp.float32),
            pltpu.VMEM((_CHUNK, D), jnp.float32),
            pltpu.SemaphoreType.DMA,
            pltpu.SemaphoreType.DMA,
            pltpu.SemaphoreType.DMA,
            pltpu.SemaphoreType.DMA,
        ],
    )
    def gather_kernel(idx_hbm, table_hbm, out_hbm, idx_v,
                      buf0, buf1, g0, g1, s0, s1):
        wid = lax.axis_index("s") * NC + lax.axis_index("c")
        base_chunk = wid * n_chunks
        pltpu.sync_copy(idx_hbm.at[pl.ds(base_chunk, n_chunks)], idx_v)
        bufs = (buf0, buf1)
        gsems = (g0, g1)
        ssems = (s0, s1)

        def out_slice(i):
            return out_hbm.at[pl.ds((base_chunk + i) * _CHUNK, _CHUNK)]

        for b in range(2):
            pltpu.async_copy(table_hbm.at[idx_v.at[b]], bufs[b], gsems[b]).wait()
            pltpu.async_copy(bufs[b], out_slice(b), ssems[b])

        def body(grp, carry):
            for b in range(2):
                i = 2 + 2 * grp + b
                pltpu.make_async_copy(bufs[b], out_slice(0), ssems[b]).wait()
                pltpu.async_copy(table_hbm.at[idx_v.at[i]], bufs[b], gsems[b]).wait()
                pltpu.async_copy(bufs[b], out_slice(i), ssems[b])
            return carry

        lax.fori_loop(0, (n_chunks - 2) // 2, body, 0)

        for b in range(2):
            pltpu.make_async_copy(bufs[b], out_slice(0), ssems[b]).wait()

    return gather_kernel(idx2d, table)


def kernel(tokens, start_pos, tok_embeddings_weight):
    B, S = tokens.shape
    V, D = tok_embeddings_weight.shape
    out = _embedding_lookup(tokens.reshape(B * S), tok_embeddings_weight)
    return out.reshape(B, S, D)


# speedup vs baseline: 1.7653x; 1.0325x over previous
import functools

import jax
import jax.numpy as jnp
from jax import lax
from jax.experimental import pallas as pl
from jax.experimental.pallas import tpu as pltpu
from jax.experimental.pallas import tpu_sc as plsc

_CHUNK = 16
_NBUF = 4


def _embedding_lookup(tokens_flat, table):
    B, = tokens_flat.shape
    V, D = table.shape
    info = plsc.get_sparse_core_info()
    NC, NS = info.num_cores, info.num_subcores
    NW = NC * NS
    b_per_w = B // NW
    n_chunks = b_per_w // _CHUNK
    assert B == NW * b_per_w and b_per_w == n_chunks * _CHUNK

    idx2d = tokens_flat.reshape(B // _CHUNK, _CHUNK)
    mesh = plsc.VectorSubcoreMesh(core_axis_name="c", subcore_axis_name="s")

    @functools.partial(
        pl.kernel,
        mesh=mesh,
        out_type=jax.ShapeDtypeStruct((B, D), jnp.float32),
        scratch_types=[
            pltpu.VMEM((n_chunks, _CHUNK), jnp.int32),
        ]
        + [pltpu.VMEM((_CHUNK, D), jnp.float32)] * _NBUF
        + [pltpu.SemaphoreType.DMA] * (2 * _NBUF),
    )
    def gather_kernel(idx_hbm, table_hbm, out_hbm, idx_v, *bufs_sems):
        bufs = bufs_sems[:_NBUF]
        gsems = bufs_sems[_NBUF:2 * _NBUF]
        ssems = bufs_sems[2 * _NBUF:]
        wid = lax.axis_index("s") * NC + lax.axis_index("c")
        base_chunk = wid * n_chunks
        pltpu.sync_copy(idx_hbm.at[pl.ds(base_chunk, n_chunks)], idx_v)

        def out_slice(i):
            return out_hbm.at[pl.ds((base_chunk + i) * _CHUNK, _CHUNK)]

        def start_gather(i, b):
            pltpu.async_copy(table_hbm.at[idx_v.at[i]], bufs[b], gsems[b])

        for b in range(_NBUF - 1):
            start_gather(b, b)

        def step(i, b, first, last):
            pltpu.make_async_copy(table_hbm.at[idx_v.at[0]], bufs[b],
                                  gsems[b]).wait()
            pltpu.async_copy(bufs[b], out_slice(i), ssems[b])
            if not last:
                nb = (b + _NBUF - 1) % _NBUF
                if not first:
                    pltpu.make_async_copy(bufs[nb], out_slice(0),
                                          ssems[nb]).wait()
                start_gather(i + _NBUF - 1, nb)

        step(0, 0, first=True, last=False)

        def body(grp, carry):
            for k in range(_NBUF):
                i = 1 + _NBUF * grp + k
                step(i, (1 + k) % _NBUF, first=False, last=False)
            return carry

        n_steady = (n_chunks - 1 - (_NBUF - 1)) // _NBUF
        lax.fori_loop(0, n_steady, body, 0)

        for k in range(_NBUF - 1):
            i = n_chunks - (_NBUF - 1) + k
            step(i, i % _NBUF, first=False, last=True)

        for b in range(_NBUF):
            pltpu.make_async_copy(bufs[b], out_slice(0), ssems[b]).wait()

    return gather_kernel(idx2d, table)


def kernel(tokens, start_pos, tok_embeddings_weight):
    B, S = tokens.shape
    V, D = tok_embeddings_weight.shape
    out = _embedding_lookup(tokens.reshape(B * S), tok_embeddings_weight)
    return out.reshape(B, S, D)
